# Initial kernel scaffold; baseline (speedup 1.0000x reference)
import jax
import jax.numpy as jnp
from jax.experimental import pallas as pl


def kernel(mem, idx, val):
    raise NotImplementedError("write your pallas kernel here")


# speedup vs baseline: 7.3173x; 7.3173x over previous
import functools

import jax
import jax.numpy as jnp
from jax import lax
from jax.experimental import pallas as pl
from jax.experimental.pallas import tpu as pltpu
from jax.experimental.pallas import tpu_sc as plsc

M, D, B = 1_000_000, 32, 16384
L = 16
NC, NS = 2, 16
NW = NC * NS
RANGE = (M // NW) // 128 * 128
TAIL = M - NW * RANGE
TAILP = 640
WCAP = RANGE + TAILP
CH = 2048
NCHI = B // CH
CW = 1024
NFULL = RANGE // CW
REM = RANGE - NFULL * CW

_mesh = plsc.VectorSubcoreMesh(core_axis_name="c", subcore_axis_name="s")


@functools.partial(
    pl.kernel,
    out_type=jax.ShapeDtypeStruct((D, M), jnp.float32),
    mesh=_mesh,
    compiler_params=pltpu.CompilerParams(needs_layout_passes=False),
    scratch_types=[
        pltpu.VMEM((WCAP,), jnp.int32),
        pltpu.VMEM((CH,), jnp.int32),
        pltpu.VMEM((CW + L,), jnp.int32),
        pltpu.VMEM((CW + L,), jnp.int32),
        pltpu.VMEM((D, CW), jnp.float32),
        pltpu.VMEM((D, CW), jnp.float32),
        pltpu.VMEM((L, 8, D), jnp.float32),
        pltpu.SemaphoreType.DMA,
        pltpu.SemaphoreType.DMA,
        pltpu.SemaphoreType.DMA,
        pltpu.SemaphoreType.DMA,
        pltpu.SemaphoreType.DMA,
    ],
)
def _sc_scatter_overwrite(memT, idx, val, outT,
                          w_ref, idxb, lrow, lpos, bufa, bufb, vfetch,
                          ina_sem, outa_sem, inb_sem, outb_sem, fsem):
    c = lax.axis_index("c")
    s = lax.axis_index("s")
    wid = s * NC + c
    lo = wid * RANGE
    ncols = jnp.where(wid == NW - 1, RANGE + TAIL, RANGE)
    iota = lax.iota(jnp.int32, L)

    neg1 = jnp.full((L,), -1, jnp.int32)

    def init_body(i, _):
        w_ref[pl.ds(i * L, L)] = neg1
        return 0

    lax.fori_loop(0, WCAP // L, init_body, 0)

    for cidx in range(NCHI):
        pltpu.sync_copy(idx.at[pl.ds(cidx * CH, CH)], idxb)

        def seed_body(j, conf, cidx=cidx):
            v = idxb[pl.ds(j * L, L)]
            pos = cidx * CH + j * L + iota
            rel = v - lo
            mask = (rel >= 0) & (rel < ncols)
            rel_s = jnp.where(mask, rel, 0)
            plsc.store_scatter(w_ref, [rel_s], pos, mask=mask)
            g = plsc.load_gather(w_ref, [rel_s])
            bad = mask & (g != pos)
            return conf + jnp.max(plsc.all_reduce_population_count(bad))

        conf = lax.fori_loop(0, CH // L, seed_body, jnp.int32(0))

        @pl.when(conf > 0)
        def _fix(cidx=cidx):
            def fix_pass(n):
                def fb(j, acc):
                    v = idxb[pl.ds(j * L, L)]
                    pos = cidx * CH + j * L + iota
                    rel = v - lo
                    mask = (rel >= 0) & (rel < ncols)
                    rel_s = jnp.where(mask, rel, 0)
                    g = plsc.load_gather(w_ref, [rel_s])
                    need = mask & (g < pos)
                    plsc.store_scatter(w_ref, [rel_s], pos, mask=need)
                    return acc + jnp.max(plsc.all_reduce_population_count(need))
                return lax.fori_loop(0, CH // L, fb, jnp.int32(0))
            lax.while_loop(lambda n: n > 0, fix_pass, jnp.int32(1))

    def fire_in(wrel, width, buf, sem):
        return pltpu.async_copy(
            memT.at[:, pl.ds(lo + wrel, width)], buf.at[:, pl.ds(0, width)],
            sem)

    def fire_out(wrel, width, buf, sem):
        return pltpu.async_copy(
            buf.at[:, pl.ds(0, width)], outT.at[:, pl.ds(lo + wrel, width)],
            sem)

    def wait_in(width, buf, sem):
        pltpu.make_async_copy(
            memT.at[:, pl.ds(lo, width)], buf.at[:, pl.ds(0, width)],
            sem).wait()

    def wait_out(width, buf, sem):
        pltpu.make_async_copy(
            buf.at[:, pl.ds(0, width)], outT.at[:, pl.ds(lo, width)],
            sem).wait()

    def process_window(wrel, width, buf):
        def scanv(i, nw):
            wv = w_ref[pl.ds(wrel + i * L, L)]
            m = wv >= 0
            plsc.store_compressed(lrow.at[pl.ds(nw, L)], i * L + iota, mask=m)
            plsc.store_compressed(lpos.at[pl.ds(nw, L)], wv, mask=m)
            return nw + jnp.max(plsc.all_reduce_population_count(m))

        nw = lax.fori_loop(0, width // L, scanv, jnp.int32(0))

        @pl.when(nw > 0)
        def _apply():
            lastr = plsc.load_gather(lrow, [jnp.full((L,), nw - 1, jnp.int32)])
            lastp = plsc.load_gather(lpos, [jnp.full((L,), nw - 1, jnp.int32)])
            lrow[pl.ds(nw, L)] = lastr
            lpos[pl.ds(nw, L)] = lastp

            def qbody(q, _):
                lposv = lpos[pl.ds(q * L, L)]
                lrowv = lrow[pl.ds(q * L, L)]
                lgrp = (lposv // 8) * 8
                hs = []
                for k in range(L):
                    gk = pl.multiple_of(lgrp[k], 8)
                    hs.append(pltpu.async_copy(
                        val.at[pl.ds(gk, 8), :], vfetch.at[k], fsem))
                for h in hs:
                    h.wait()
                pmod = lposv - (lposv // 8) * 8
                for d in range(D):
                    dsplat = jnp.full((L,), d, jnp.int32)
                    data = plsc.load_gather(vfetch, [iota, pmod, dsplat])
                    plsc.store_scatter(buf, [dsplat, lrowv], data)
                return 0

            lax.fori_loop(0, (nw + L - 1) // L, qbody, 0)

    fire_in(0, CW, bufa, ina_sem)
    fire_in(CW, CW, bufb, inb_sem)

    def pipe_body(t, _):
        wa = (2 * t) * CW
        wb = (2 * t + 1) * CW
        wait_in(CW, bufa, ina_sem)
        process_window(wa, CW, bufa)
        fire_out(wa, CW, bufa, outa_sem)
        wait_in(CW, bufb, inb_sem)
        process_window(wb, CW, bufb)
        fire_out(wb, CW, bufb, outb_sem)
        wait_out(CW, bufa, outa_sem)
        wait_out(CW, bufb, outb_sem)

        @pl.when(t < NFULL // 2 - 1)
        def _prefetch():
            fire_in(wa + 2 * CW, CW, bufa, ina_sem)
            fire_in(wb + 2 * CW, CW, bufb, inb_sem)
        return 0

    lax.fori_loop(0, NFULL // 2, pipe_body, 0)

    fire_in(NFULL * CW, REM, bufa, ina_sem)
    wait_in(REM, bufa, ina_sem)
    process_window(NFULL * CW, REM, bufa)
    fire_out(NFULL * CW, REM, bufa, outa_sem)
    wait_out(REM, bufa, outa_sem)

    @pl.when(wid == NW - 1)
    def _tail():
        fire_in(RANGE, TAILP, bufa, ina_sem)
        wait_in(TAILP, bufa, ina_sem)
        process_window(RANGE, TAILP, bufa)
        fire_out(RANGE, TAILP, bufa, outa_sem)
        wait_out(TAILP, bufa, outa_sem)


def kernel(mem, idx, val):
    outT = _sc_scatter_overwrite(mem.T, idx, val)
    return outT.T
